# baseline (speedup 1.0000x reference)
import jax
import jax.numpy as jnp
from jax import lax
from jax.experimental import pallas as pl
from jax.experimental.pallas import tpu as pltpu
from jax.experimental.pallas import tpu_sc as plsc

_N = 1024
_L = 16
_NT = 16
_CPT = _N // _NT
_NG = _CPT // _L
_NCH = _N // _L
_PW = _CPT + _CPT + _L


def _sc_body(adj_hbm, src_hbm, emb_hbm, nf_out, stats_out,
             blk, pub, pball, cntb, srcv, statv, chg, itc,
             rowbuf, sh_pub):
    t = lax.axis_index("s")
    col0 = t * _CPT
    iot = lax.iota(jnp.int32, _L)
    nf = emb_hbm.shape[1]
    inf_v = jnp.full((_L,), jnp.inf, dtype=jnp.float32)

    def f2i(x):
        return plsc.bitcast(x, jnp.int32)

    def i2f(x):
        return plsc.bitcast(x, jnp.float32)

    pltpu.sync_copy(adj_hbm.at[:, pl.ds(col0, _CPT)], blk)
    pltpu.sync_copy(src_hbm, srcv)
    pltpu.sync_copy(emb_hbm.at[pl.ds(col0, _CPT), :],
                    rowbuf.at[:, pl.ds(0, nf)])
    src_splat = srcv[...]
    srow_splat = src_splat >> 6

    def diag_body(i, _):
        row = col0 + i
        coff = (i // _L) * _L
        v = blk[row, pl.ds(coff, _L)]
        blk[row, pl.ds(coff, _L)] = jnp.where(iot == i % _L, jnp.inf, v)
        return 0
    lax.fori_loop(0, _CPT, diag_body, 0)

    def pinit_body(r, _):
        for g in range(_NG):
            gidx = r * _CPT + g * _L + iot
            pball[r, pl.ds(g * _L, _L)] = f2i(
                jnp.where(gidx == src_splat, 0.0, jnp.inf))
        pball[r, pl.ds(_CPT, _L)] = jnp.where(iot == 0, src_splat, 0)
        return 0
    lax.fori_loop(0, _NT, pinit_body, 0)
    cntb[...] = jnp.where(iot == srow_splat, 1, 0)
    for g in range(_NG + 1):
        pub[pl.ds(_CPT + g * _L, _L)] = jnp.zeros((_L,), jnp.int32)
    chg[0] = jnp.int32(1)
    itc[0] = jnp.int32(0)
    itc[1] = jnp.int32(0)

    def relax_iter():
        cv = cntb[...]
        accs = [inf_v] * _NG

        for r in range(_NT):
            cnt = cv[r]
            nchf = (cnt + (_L - 1)) // _L
            itc[1] = itc[1] + nchf

            def f_step(ci, accs_t, r=r):
                idxv = pball[r, pl.ds(_CPT + ci * _L, _L)]
                dv = i2f(plsc.load_gather(pball, [idxv >> 6, idxv & 63]))
                out = list(accs_t)
                for j in range(_L):
                    s = idxv[j]
                    a = jnp.full((_L,), dv[j])
                    for g in range(_NG):
                        cand = blk[s, pl.ds(g * _L, _L)] + a
                        out[g] = jnp.minimum(out[g], cand)
                return tuple(out)
            accs = list(lax.fori_loop(0, nchf, f_step, tuple(accs)))

        off = jnp.int32(0)
        for g in range(_NG):
            cur = i2f(pball[t, pl.ds(g * _L, _L)])
            m = accs[g] < cur
            pub[pl.ds(g * _L, _L)] = f2i(jnp.minimum(accs[g], cur))
            plsc.store_compressed(pub.at[pl.ds(_CPT + off, _L)],
                                  col0 + g * _L + iot, mask=m)
            pc = plsc.all_reduce_population_count(m)
            off = off + pc[0]
        pub[pl.ds(2 * _CPT, _L)] = jnp.full((_L,), 1, jnp.int32) * off

        pltpu.sync_copy(pub, sh_pub.at[t])
        plsc.subcore_barrier()
        pltpu.sync_copy(sh_pub, pball)

        cnew = plsc.load_gather(
            pball, [iot, jnp.full((_L,), 2 * _CPT, jnp.int32)])
        cntb[...] = cnew
        chg[0] = (jnp.max(cnew) > 0).astype(jnp.int32)
        itc[0] = itc[0] + 1

        plsc.subcore_barrier()

    def inner_body(i, _):
        @pl.when(chg[0] > 0)
        def _():
            relax_iter()
        return 0

    def outer_body(o, _):
        @pl.when(chg[0] > 0)
        def _():
            lax.fori_loop(0, 16, inner_body, 0)
        return 0

    lax.fori_loop(0, 63, outer_body, 0)

    @pl.when(chg[0] > 0)
    def _():
        lax.fori_loop(0, 15, inner_body, 0)

    col_idx = jnp.full((_L,), nf, dtype=jnp.int32)
    for g in range(_NG):
        vals = i2f(pball[t, pl.ds(g * _L, _L)])
        plsc.store_scatter(rowbuf, [g * _L + iot, col_idx], vals)
    pltpu.sync_copy(rowbuf, nf_out.at[pl.ds(col0, _CPT), :])

    @pl.when(t == 0)
    def _():
        def stat_body(k, acc):
            d = i2f(pball[k // _NG, pl.ds((k % _NG) * _L, _L)])
            gidx = k * _L + iot
            return (jnp.maximum(acc[0], d),
                    acc[1] + jnp.where(gidx == src_splat, d, 0.0))
        dm, ec = lax.fori_loop(
            0, _NCH, stat_body,
            (jnp.full((_L,), -jnp.inf, dtype=jnp.float32),
             jnp.zeros((_L,), jnp.float32)))
        diam = jnp.max(dm)
        ecc = jnp.sum(ec)
        statv[...] = jnp.where(
            iot == 0, diam,
            jnp.where(iot == 1, ecc,
                      jnp.where(iot == 2, itc[0].astype(jnp.float32),
                                itc[1].astype(jnp.float32))))
        pltpu.sync_copy(statv, stats_out)


def _run_sc(adj_matrix, src_arr, emb):
    n = adj_matrix.shape[0]
    nf = emb.shape[1]
    mesh = plsc.VectorSubcoreMesh(core_axis_name="c", subcore_axis_name="s",
                                  num_cores=1)
    sc = pl.kernel(
        _sc_body,
        out_type=(jax.ShapeDtypeStruct((n, nf + 1), jnp.float32),
                  jax.ShapeDtypeStruct((_L,), jnp.float32)),
        mesh=mesh,
        scratch_types=[
            pltpu.VMEM((n, _CPT), jnp.float32),
            pltpu.VMEM((_PW,), jnp.int32),
            pltpu.VMEM((_NT, _PW), jnp.int32),
            pltpu.VMEM((_L,), jnp.int32),
            pltpu.VMEM((_L,), jnp.int32),
            pltpu.VMEM((_L,), jnp.float32),
            pltpu.SMEM((1,), jnp.int32),
            pltpu.SMEM((2,), jnp.int32),
            pltpu.VMEM((_CPT, nf + 1), jnp.float32),
            pltpu.VMEM_SHARED((_NT, _PW), jnp.int32),
        ],
        compiler_params=pltpu.CompilerParams(use_tc_tiling_on_sc=False,
                                             needs_layout_passes=False),
    )
    return sc(adj_matrix, src_arr, emb)


def kernel(adj_matrix, source_node, emb, edge_weights):
    src_arr = jnp.full((_L,), source_node, dtype=jnp.int32)
    node_features, stats = _run_sc(adj_matrix, src_arr, emb)
    return node_features, stats[0], stats[1]


# speedup vs baseline: 1.1858x; 1.1858x over previous
import jax
import jax.numpy as jnp
from jax import lax
from jax.experimental import pallas as pl
from jax.experimental.pallas import tpu as pltpu
from jax.experimental.pallas import tpu_sc as plsc

_N = 1024
_L = 16
_NT = 16
_CPT = _N // _NT
_NG = _CPT // _L
_NCH = _N // _L


def _sc_body(adj_hbm, src_hbm, emb_hbm, nf_out, stats_out,
             blk, dist, newd, myout, srcv, statv, chgidx, chg, mcnt,
             rowbuf, sh_dist):
    c = lax.axis_index("c")
    t = lax.axis_index("s")
    col0 = t * _CPT
    iot = lax.iota(jnp.int32, _L)
    nf = emb_hbm.shape[1]

    pltpu.sync_copy(adj_hbm.at[:, pl.ds(col0, _CPT)], blk)
    pltpu.sync_copy(src_hbm, srcv)

    @pl.when(c == 0)
    def _():
        pltpu.sync_copy(emb_hbm.at[pl.ds(col0, _CPT), :],
                        rowbuf.at[:, pl.ds(0, nf)])
    src_splat = srcv[...]

    inf_v = jnp.full((_L,), jnp.inf, dtype=jnp.float32)

    def diag_body(i, _):
        row = col0 + i
        coff = (i // _L) * _L
        v = blk[row, pl.ds(coff, _L)]
        blk[row, pl.ds(coff, _L)] = jnp.where(iot == i % _L, jnp.inf, v)
        return 0
    lax.fori_loop(0, _CPT, diag_body, 0)

    def init_body(k, _):
        gidx = iot + k * _L
        dist[pl.ds(k * _L, _L)] = jnp.where(gidx == src_splat, 0.0, jnp.inf)
        chgidx[pl.ds(k * _L, _L)] = jnp.zeros((_L,), jnp.int32)
        return 0
    lax.fori_loop(0, _NCH, init_body, 0)
    chgidx[pl.ds(0, _L)] = jnp.where(iot == 0, src_splat, 0)
    mcnt[0] = jnp.int32(1)

    chg[0] = jnp.int32(1)

    def relax_iter():
        nch = (mcnt[0] + (_L - 1)) // _L

        def c_step(ci, accs):
            idxv = chgidx[pl.ds(ci * _L, _L)]
            dv = plsc.load_gather(dist, [idxv])
            out = list(accs)
            for j in range(_L):
                s = idxv[j]
                a = jnp.full((_L,), dv[j])
                for g in range(_NG):
                    cand = blk[s, pl.ds(g * _L, _L)] + a
                    out[g] = jnp.minimum(out[g], cand)
            return tuple(out)
        accs = lax.fori_loop(0, nch, c_step, (inf_v,) * _NG)

        for g in range(_NG):
            cur = dist[pl.ds(col0 + g * _L, _L)]
            myout[pl.ds(g * _L, _L)] = jnp.minimum(accs[g], cur)

        pltpu.sync_copy(myout, sh_dist.at[pl.ds(col0, _CPT)])
        plsc.subcore_barrier()
        pltpu.sync_copy(sh_dist, newd)

        def ch_body(k, off):
            o = dist[pl.ds(k * _L, _L)]
            nv = newd[pl.ds(k * _L, _L)]
            dist[pl.ds(k * _L, _L)] = nv
            m = nv < o
            plsc.store_compressed(chgidx.at[pl.ds(off, _L)], iot + k * _L,
                                  mask=m)
            pc = plsc.all_reduce_population_count(m)
            return off + pc[0]
        off = lax.fori_loop(0, _NCH, ch_body, jnp.int32(0))
        mcnt[0] = off
        chg[0] = (off > 0).astype(jnp.int32)

        plsc.subcore_barrier()

    def inner_body(i, _):
        @pl.when(chg[0] > 0)
        def _():
            relax_iter()
        return 0

    def outer_body(o, _):
        @pl.when(chg[0] > 0)
        def _():
            lax.fori_loop(0, 16, inner_body, 0)
        return 0

    lax.fori_loop(0, 63, outer_body, 0)

    @pl.when(chg[0] > 0)
    def _():
        lax.fori_loop(0, 15, inner_body, 0)

    @pl.when(c == 0)
    def _():
        col_idx = jnp.full((_L,), nf, dtype=jnp.int32)
        for g in range(_NG):
            vals = dist[pl.ds(col0 + g * _L, _L)]
            plsc.store_scatter(rowbuf, [g * _L + iot, col_idx], vals)
        pltpu.sync_copy(rowbuf, nf_out.at[pl.ds(col0, _CPT), :])

    @pl.when(jnp.logical_and(c == 0, t == 0))
    def _():
        def stat_body(k, acc):
            d = dist[pl.ds(k * _L, _L)]
            gidx = iot + k * _L
            return (jnp.maximum(acc[0], d),
                    acc[1] + jnp.where(gidx == src_splat, d, 0.0))
        dm, ec = lax.fori_loop(
            0, _NCH, stat_body,
            (jnp.full((_L,), -jnp.inf, dtype=jnp.float32),
             jnp.zeros((_L,), jnp.float32)))
        diam = jnp.max(dm)
        ecc = jnp.sum(ec)
        statv[...] = jnp.where(iot == 0, diam, jnp.where(iot == 1, ecc, 0.0))
        pltpu.sync_copy(statv, stats_out)


def _run_sc(adj_matrix, src_arr, emb):
    n = adj_matrix.shape[0]
    nf = emb.shape[1]
    mesh = plsc.VectorSubcoreMesh(core_axis_name="c", subcore_axis_name="s",
                                  num_cores=1)
    sc = pl.kernel(
        _sc_body,
        out_type=(jax.ShapeDtypeStruct((n, nf + 1), jnp.float32),
                  jax.ShapeDtypeStruct((_L,), jnp.float32)),
        mesh=mesh,
        scratch_types=[
            pltpu.VMEM((n, _CPT), jnp.float32),
            pltpu.VMEM((n,), jnp.float32),
            pltpu.VMEM((n,), jnp.float32),
            pltpu.VMEM((_CPT,), jnp.float32),
            pltpu.VMEM((_L,), jnp.int32),
            pltpu.VMEM((_L,), jnp.float32),
            pltpu.VMEM((n,), jnp.int32),
            pltpu.SMEM((1,), jnp.int32),
            pltpu.SMEM((1,), jnp.int32),
            pltpu.VMEM((_CPT, nf + 1), jnp.float32),
            pltpu.VMEM_SHARED((n,), jnp.float32),
        ],
        compiler_params=pltpu.CompilerParams(use_tc_tiling_on_sc=False,
                                             needs_layout_passes=False),
    )
    return sc(adj_matrix, src_arr, emb)


def kernel(adj_matrix, source_node, emb, edge_weights):
    src_arr = jnp.full((_L,), source_node, dtype=jnp.int32)
    node_features, stats = _run_sc(adj_matrix, src_arr, emb)
    return node_features, stats[0], stats[1]


# speedup vs baseline: 1.2070x; 1.0179x over previous
import jax
import jax.numpy as jnp
from jax import lax
from jax.experimental import pallas as pl
from jax.experimental.pallas import tpu as pltpu
from jax.experimental.pallas import tpu_sc as plsc

_N = 1024
_L = 16
_NT = 16
_CPT = _N // _NT
_NG = _CPT // _L
_NCH = _N // _L


def _sc_body(adj_hbm, src_hbm, emb_hbm, nf_out, stats_out,
             blk, dist, newd, myout, srcv, statv, chgidx, chg, mcnt,
             rowbuf, sh_dist):
    c = lax.axis_index("c")
    t = lax.axis_index("s")
    col0 = t * _CPT
    iot = lax.iota(jnp.int32, _L)
    nf = emb_hbm.shape[1]

    pltpu.sync_copy(adj_hbm.at[:, pl.ds(col0, _CPT)], blk)
    pltpu.sync_copy(src_hbm, srcv)

    @pl.when(c == 0)
    def _():
        pltpu.sync_copy(emb_hbm.at[pl.ds(col0, _CPT), :],
                        rowbuf.at[:, pl.ds(0, nf)])
    src_splat = srcv[...]

    inf_v = jnp.full((_L,), jnp.inf, dtype=jnp.float32)

    def diag_body(i, _):
        row = col0 + i
        coff = (i // _L) * _L
        v = blk[row, pl.ds(coff, _L)]
        blk[row, pl.ds(coff, _L)] = jnp.where(iot == i % _L, jnp.inf, v)
        return 0
    lax.fori_loop(0, _CPT, diag_body, 0)

    def init_body(k, _):
        gidx = iot + k * _L
        dist[pl.ds(k * _L, _L)] = jnp.where(gidx == src_splat, 0.0, jnp.inf)
        chgidx[pl.ds(k * _L, _L)] = jnp.zeros((_L,), jnp.int32)
        return 0
    lax.fori_loop(0, _NCH, init_body, 0)
    chgidx[pl.ds(0, _L)] = jnp.where(iot == 0, src_splat, 0)
    mcnt[0] = jnp.int32(1)

    chg[0] = jnp.int32(1)

    def relax_iter():
        nch = (mcnt[0] + (_L - 1)) // _L

        def c_step(ci, accs):
            idxv = chgidx[pl.ds(ci * _L, _L)]
            dv = plsc.load_gather(dist, [idxv])
            out = list(accs)
            for j in range(_L):
                s = idxv[j]
                a = jnp.full((_L,), dv[j])
                for g in range(_NG):
                    cand = blk[s, pl.ds(g * _L, _L)] + a
                    out[g] = jnp.minimum(out[g], cand)
            return tuple(out)
        accs = lax.fori_loop(0, nch, c_step, (inf_v,) * _NG)

        for g in range(_NG):
            cur = dist[pl.ds(col0 + g * _L, _L)]
            myout[pl.ds(g * _L, _L)] = jnp.minimum(accs[g], cur)

        pltpu.sync_copy(myout, sh_dist.at[pl.ds(col0, _CPT)])
        plsc.subcore_barrier()
        pltpu.sync_copy(sh_dist, newd)

        def ch_body(k4, off):
            for u in range(4):
                k = k4 * 4 + u
                o = dist[pl.ds(k * _L, _L)]
                nv = newd[pl.ds(k * _L, _L)]
                dist[pl.ds(k * _L, _L)] = nv
                m = nv < o
                plsc.store_compressed(chgidx.at[pl.ds(off, _L)],
                                      iot + k * _L, mask=m)
                pc = plsc.all_reduce_population_count(m)
                off = off + pc[0]
            return off
        off = lax.fori_loop(0, _NCH // 4, ch_body, jnp.int32(0))
        mcnt[0] = off
        chg[0] = (off > 0).astype(jnp.int32)

        plsc.subcore_barrier()

    def inner_body(i, _):
        @pl.when(chg[0] > 0)
        def _():
            relax_iter()
        return 0

    def outer_body(o, _):
        @pl.when(chg[0] > 0)
        def _():
            lax.fori_loop(0, 16, inner_body, 0)
        return 0

    lax.fori_loop(0, 63, outer_body, 0)

    @pl.when(chg[0] > 0)
    def _():
        lax.fori_loop(0, 15, inner_body, 0)

    @pl.when(c == 0)
    def _():
        col_idx = jnp.full((_L,), nf, dtype=jnp.int32)
        for g in range(_NG):
            vals = dist[pl.ds(col0 + g * _L, _L)]
            plsc.store_scatter(rowbuf, [g * _L + iot, col_idx], vals)
        pltpu.sync_copy(rowbuf, nf_out.at[pl.ds(col0, _CPT), :])

    @pl.when(jnp.logical_and(c == 0, t == 0))
    def _():
        def stat_body(k, acc):
            d = dist[pl.ds(k * _L, _L)]
            gidx = iot + k * _L
            return (jnp.maximum(acc[0], d),
                    acc[1] + jnp.where(gidx == src_splat, d, 0.0))
        dm, ec = lax.fori_loop(
            0, _NCH, stat_body,
            (jnp.full((_L,), -jnp.inf, dtype=jnp.float32),
             jnp.zeros((_L,), jnp.float32)))
        diam = jnp.max(dm)
        ecc = jnp.sum(ec)
        statv[...] = jnp.where(iot == 0, diam, jnp.where(iot == 1, ecc, 0.0))
        pltpu.sync_copy(statv, stats_out)


def _run_sc(adj_matrix, src_arr, emb):
    n = adj_matrix.shape[0]
    nf = emb.shape[1]
    mesh = plsc.VectorSubcoreMesh(core_axis_name="c", subcore_axis_name="s",
                                  num_cores=1)
    sc = pl.kernel(
        _sc_body,
        out_type=(jax.ShapeDtypeStruct((n, nf + 1), jnp.float32),
                  jax.ShapeDtypeStruct((_L,), jnp.float32)),
        mesh=mesh,
        scratch_types=[
            pltpu.VMEM((n, _CPT), jnp.float32),
            pltpu.VMEM((n,), jnp.float32),
            pltpu.VMEM((n,), jnp.float32),
            pltpu.VMEM((_CPT,), jnp.float32),
            pltpu.VMEM((_L,), jnp.int32),
            pltpu.VMEM((_L,), jnp.float32),
            pltpu.VMEM((n,), jnp.int32),
            pltpu.SMEM((1,), jnp.int32),
            pltpu.SMEM((1,), jnp.int32),
            pltpu.VMEM((_CPT, nf + 1), jnp.float32),
            pltpu.VMEM_SHARED((n,), jnp.float32),
        ],
        compiler_params=pltpu.CompilerParams(use_tc_tiling_on_sc=False,
                                             needs_layout_passes=False),
    )
    return sc(adj_matrix, src_arr, emb)


def kernel(adj_matrix, source_node, emb, edge_weights):
    src_arr = jnp.full((_L,), source_node, dtype=jnp.int32)
    node_features, stats = _run_sc(adj_matrix, src_arr, emb)
    return node_features, stats[0], stats[1]
